# baseline (speedup 1.0000x reference)
import jax
import jax.numpy as jnp
from jax.experimental import pallas as pl
from jax.experimental.pallas import tpu as pltpu

_CHUNK = 4
_DEPTH = 4


def _chunk_schedule(b):
    if b >= 4 * _CHUNK and b % _CHUNK == 0:
        return [2, 2] + [_CHUNK] * ((b - 8) // _CHUNK) + [2, 2]
    return [_CHUNK] * (b // _CHUNK)


def _pipelined_body(a_hbm, f_hbm, w0_ref, w1_ref, o_hbm,
                    abuf, fbuf, obuf, a_sem, f_sem, o_sem):
    b, n, d = f_hbm.shape
    w0 = w0_ref[...].astype(jnp.bfloat16)
    w1 = w1_ref[...].astype(jnp.bfloat16)

    sizes = _chunk_schedule(b)
    offs = [sum(sizes[:i]) for i in range(len(sizes))]
    n_chunks = len(sizes)

    def start_in(c):
        s = c % _DEPTH
        pltpu.make_async_copy(
            f_hbm.at[pl.ds(offs[c], sizes[c])],
            fbuf.at[s, pl.ds(0, sizes[c])], f_sem.at[s]).start()
        pltpu.make_async_copy(
            a_hbm.at[pl.ds(offs[c], sizes[c])],
            abuf.at[s, pl.ds(0, sizes[c])], a_sem.at[s]).start()

    for c in range(min(_DEPTH, n_chunks)):
        start_in(c)

    for c in range(n_chunks):
        s = c % _DEPTH
        csz = sizes[c]
        pltpu.make_async_copy(
            f_hbm.at[pl.ds(0, csz)],
            fbuf.at[s, pl.ds(0, csz)], f_sem.at[s]).wait()
        pltpu.make_async_copy(
            a_hbm.at[pl.ds(0, csz)],
            abuf.at[s, pl.ds(0, csz)], a_sem.at[s]).wait()

        fb = fbuf[s, 0:csz].astype(jnp.bfloat16).reshape(csz * n, d)
        g0 = jnp.dot(fb, w0, preferred_element_type=jnp.float32)
        g0 = g0.astype(jnp.bfloat16).reshape(csz, n, d)

        a_bf = [abuf[s, g].astype(jnp.bfloat16) for g in range(csz)]
        h1 = [
            jnp.maximum(
                jnp.dot(a_bf[g], g0[g], preferred_element_type=jnp.float32),
                0.0).astype(jnp.bfloat16)
            for g in range(csz)
        ]

        g1 = jnp.dot(jnp.concatenate(h1, axis=0), w1,
                     preferred_element_type=jnp.float32)
        g1 = g1.astype(jnp.bfloat16).reshape(csz, n, d)

        if c >= _DEPTH:
            pltpu.make_async_copy(
                obuf.at[s, pl.ds(0, sizes[c - _DEPTH])],
                o_hbm.at[pl.ds(0, sizes[c - _DEPTH])], o_sem.at[s]).wait()
        for g in range(csz):
            obuf[s, g] = jnp.maximum(
                jnp.dot(a_bf[g], g1[g], preferred_element_type=jnp.float32),
                0.0)
        pltpu.make_async_copy(
            obuf.at[s, pl.ds(0, csz)],
            o_hbm.at[pl.ds(offs[c], csz)], o_sem.at[s]).start()

        if c + _DEPTH < n_chunks:
            start_in(c + _DEPTH)

    for c in range(max(0, n_chunks - _DEPTH), n_chunks):
        s = c % _DEPTH
        pltpu.make_async_copy(
            obuf.at[s, pl.ds(0, sizes[c])],
            o_hbm.at[pl.ds(0, sizes[c])], o_sem.at[s]).wait()


def _single_graph_body(a_ref, f_ref, w0_ref, w1_ref, o_ref):
    a = a_ref[...].astype(jnp.bfloat16)
    f = f_ref[...].astype(jnp.bfloat16)
    w0 = w0_ref[...].astype(jnp.bfloat16)
    w1 = w1_ref[...].astype(jnp.bfloat16)
    g = jnp.dot(f, w0, preferred_element_type=jnp.float32)
    h = jnp.dot(a, g.astype(jnp.bfloat16), preferred_element_type=jnp.float32)
    h = jnp.maximum(h, 0.0).astype(jnp.bfloat16)
    g = jnp.dot(h, w1, preferred_element_type=jnp.float32)
    h = jnp.dot(a, g.astype(jnp.bfloat16), preferred_element_type=jnp.float32)
    o_ref[...] = jnp.maximum(h, 0.0)


def _grid_body(a_ref, f_ref, w0_ref, w1_ref, o_ref):
    g_blk, n, d = f_ref.shape
    w0 = w0_ref[...].astype(jnp.bfloat16)
    w1 = w1_ref[...].astype(jnp.bfloat16)
    f = f_ref[...].astype(jnp.bfloat16).reshape(g_blk * n, d)
    g0 = jnp.dot(f, w0, preferred_element_type=jnp.float32)
    g0 = g0.astype(jnp.bfloat16).reshape(g_blk, n, d)
    a = [a_ref[g].astype(jnp.bfloat16) for g in range(g_blk)]
    h1 = [
        jnp.maximum(
            jnp.dot(a[g], g0[g], preferred_element_type=jnp.float32), 0.0
        ).astype(jnp.bfloat16)
        for g in range(g_blk)
    ]
    g1 = jnp.dot(jnp.concatenate(h1, axis=0), w1,
                 preferred_element_type=jnp.float32)
    g1 = g1.astype(jnp.bfloat16).reshape(g_blk, n, d)
    for g in range(g_blk):
        o_ref[g] = jnp.maximum(
            jnp.dot(a[g], g1[g], preferred_element_type=jnp.float32), 0.0)


def _batched_grid_call(a_norm, f_norm, w0, w1, gblk):
    b, n, _ = a_norm.shape
    d = w0.shape[-1]
    return pl.pallas_call(
        _grid_body,
        out_shape=jax.ShapeDtypeStruct((b, n, d), jnp.float32),
        grid=(b // gblk,),
        in_specs=[
            pl.BlockSpec((gblk, n, n), lambda i: (i, 0, 0)),
            pl.BlockSpec((gblk, n, d), lambda i: (i, 0, 0)),
            pl.BlockSpec((d, d), lambda i: (0, 0)),
            pl.BlockSpec((d, d), lambda i: (0, 0)),
        ],
        out_specs=pl.BlockSpec((gblk, n, d), lambda i: (i, 0, 0)),
        compiler_params=pltpu.CompilerParams(
            dimension_semantics=("parallel",)),
    )(a_norm, f_norm, w0, w1)


def kernel(a_norm, f_norm, w0, w1):
    d = w0.shape[-1]

    if a_norm.ndim == 2:
        n = a_norm.shape[0]
        vmem = pl.BlockSpec(memory_space=pltpu.MemorySpace.VMEM)
        return pl.pallas_call(
            _single_graph_body,
            out_shape=jax.ShapeDtypeStruct((n, d), jnp.float32),
            in_specs=[vmem, vmem, vmem, vmem],
            out_specs=vmem,
        )(a_norm, f_norm, w0, w1)

    b, n, _ = a_norm.shape
    if b % _CHUNK != 0 or b < _CHUNK * _DEPTH:
        return _batched_grid_call(a_norm, f_norm, w0, w1,
                                  8 if b % 8 == 0 else 1)

    any_spec = pl.BlockSpec(memory_space=pltpu.MemorySpace.HBM)
    vmem = pl.BlockSpec(memory_space=pltpu.MemorySpace.VMEM)
    return pl.pallas_call(
        _pipelined_body,
        out_shape=jax.ShapeDtypeStruct((b, n, d), jnp.float32),
        in_specs=[any_spec, any_spec, vmem, vmem],
        out_specs=any_spec,
        scratch_shapes=[
            pltpu.VMEM((_DEPTH, _CHUNK, n, n), jnp.float32),
            pltpu.VMEM((_DEPTH, _CHUNK, n, d), jnp.float32),
            pltpu.VMEM((_DEPTH, _CHUNK, n, d), jnp.float32),
            pltpu.SemaphoreType.DMA((_DEPTH,)),
            pltpu.SemaphoreType.DMA((_DEPTH,)),
            pltpu.SemaphoreType.DMA((_DEPTH,)),
        ],
    )(a_norm, f_norm, w0, w1)


# speedup vs baseline: 1.0504x; 1.0504x over previous
import jax
import jax.numpy as jnp
from jax.experimental import pallas as pl
from jax.experimental.pallas import tpu as pltpu

_CHUNK = 4
_DEPTH = 4


def _pipelined_body(a_hbm, f_hbm, w0_ref, w1_ref, o_hbm,
                    abuf, fbuf, obuf, a_sem, f_sem, o_sem):
    b, n, d = f_hbm.shape
    n_chunks = b // _CHUNK
    w0 = w0_ref[...].astype(jnp.bfloat16)
    w1 = w1_ref[...].astype(jnp.bfloat16)

    def start_in(c):
        s = c % _DEPTH
        pltpu.make_async_copy(
            f_hbm.at[pl.ds(c * _CHUNK, _CHUNK)], fbuf.at[s], f_sem.at[s]
        ).start()
        pltpu.make_async_copy(
            a_hbm.at[pl.ds(c * _CHUNK, _CHUNK)], abuf.at[s], a_sem.at[s]
        ).start()

    for c in range(min(_DEPTH, n_chunks)):
        start_in(c)

    for c in range(n_chunks):
        s = c % _DEPTH
        pltpu.make_async_copy(
            f_hbm.at[pl.ds(0, _CHUNK)], fbuf.at[s], f_sem.at[s]).wait()
        pltpu.make_async_copy(
            a_hbm.at[pl.ds(0, _CHUNK)], abuf.at[s], a_sem.at[s]).wait()

        fb = fbuf[s].astype(jnp.bfloat16).reshape(_CHUNK * n, d)
        g0 = jnp.dot(fb, w0, preferred_element_type=jnp.float32)
        g0 = g0.astype(jnp.bfloat16).reshape(_CHUNK, n, d)

        a_bf = [abuf[s, g].astype(jnp.bfloat16) for g in range(_CHUNK)]
        h1 = [
            jnp.maximum(
                jnp.dot(a_bf[g], g0[g], preferred_element_type=jnp.float32),
                0.0).astype(jnp.bfloat16)
            for g in range(_CHUNK)
        ]

        g1 = jnp.dot(jnp.concatenate(h1, axis=0), w1,
                     preferred_element_type=jnp.float32)
        g1 = g1.astype(jnp.bfloat16).reshape(_CHUNK, n, d)

        if c >= _DEPTH:
            pltpu.make_async_copy(
                obuf.at[s], o_hbm.at[pl.ds(0, _CHUNK)], o_sem.at[s]).wait()
        for g in range(_CHUNK):
            obuf[s, g] = jnp.maximum(
                jnp.dot(a_bf[g], g1[g], preferred_element_type=jnp.float32),
                0.0)
        pltpu.make_async_copy(
            obuf.at[s], o_hbm.at[pl.ds(c * _CHUNK, _CHUNK)], o_sem.at[s]
        ).start()

        if c + _DEPTH < n_chunks:
            start_in(c + _DEPTH)

    for s in range(min(_DEPTH, n_chunks)):
        pltpu.make_async_copy(
            obuf.at[s], o_hbm.at[pl.ds(0, _CHUNK)], o_sem.at[s]).wait()


def _single_graph_body(a_ref, f_ref, w0_ref, w1_ref, o_ref):
    a = a_ref[...].astype(jnp.bfloat16)
    f = f_ref[...].astype(jnp.bfloat16)
    w0 = w0_ref[...].astype(jnp.bfloat16)
    w1 = w1_ref[...].astype(jnp.bfloat16)
    g = jnp.dot(f, w0, preferred_element_type=jnp.float32)
    h = jnp.dot(a, g.astype(jnp.bfloat16), preferred_element_type=jnp.float32)
    h = jnp.maximum(h, 0.0).astype(jnp.bfloat16)
    g = jnp.dot(h, w1, preferred_element_type=jnp.float32)
    h = jnp.dot(a, g.astype(jnp.bfloat16), preferred_element_type=jnp.float32)
    o_ref[...] = jnp.maximum(h, 0.0)


def _grid_body(a_ref, f_ref, w0_ref, w1_ref, o_ref):
    g_blk, n, d = f_ref.shape
    w0 = w0_ref[...].astype(jnp.bfloat16)
    w1 = w1_ref[...].astype(jnp.bfloat16)
    f = f_ref[...].astype(jnp.bfloat16).reshape(g_blk * n, d)
    g0 = jnp.dot(f, w0, preferred_element_type=jnp.float32)
    g0 = g0.astype(jnp.bfloat16).reshape(g_blk, n, d)
    a = [a_ref[g].astype(jnp.bfloat16) for g in range(g_blk)]
    h1 = [
        jnp.maximum(
            jnp.dot(a[g], g0[g], preferred_element_type=jnp.float32), 0.0
        ).astype(jnp.bfloat16)
        for g in range(g_blk)
    ]
    g1 = jnp.dot(jnp.concatenate(h1, axis=0), w1,
                 preferred_element_type=jnp.float32)
    g1 = g1.astype(jnp.bfloat16).reshape(g_blk, n, d)
    for g in range(g_blk):
        o_ref[g] = jnp.maximum(
            jnp.dot(a[g], g1[g], preferred_element_type=jnp.float32), 0.0)


def _batched_grid_call(a_norm, f_norm, w0, w1, gblk):
    b, n, _ = a_norm.shape
    d = w0.shape[-1]
    return pl.pallas_call(
        _grid_body,
        out_shape=jax.ShapeDtypeStruct((b, n, d), jnp.float32),
        grid=(b // gblk,),
        in_specs=[
            pl.BlockSpec((gblk, n, n), lambda i: (i, 0, 0)),
            pl.BlockSpec((gblk, n, d), lambda i: (i, 0, 0)),
            pl.BlockSpec((d, d), lambda i: (0, 0)),
            pl.BlockSpec((d, d), lambda i: (0, 0)),
        ],
        out_specs=pl.BlockSpec((gblk, n, d), lambda i: (i, 0, 0)),
        compiler_params=pltpu.CompilerParams(
            dimension_semantics=("parallel",)),
    )(a_norm, f_norm, w0, w1)


def kernel(a_norm, f_norm, w0, w1):
    d = w0.shape[-1]

    if a_norm.ndim == 2:
        n = a_norm.shape[0]
        vmem = pl.BlockSpec(memory_space=pltpu.MemorySpace.VMEM)
        return pl.pallas_call(
            _single_graph_body,
            out_shape=jax.ShapeDtypeStruct((n, d), jnp.float32),
            in_specs=[vmem, vmem, vmem, vmem],
            out_specs=vmem,
        )(a_norm, f_norm, w0, w1)

    b, n, _ = a_norm.shape
    if b % _CHUNK != 0 or b < _CHUNK * _DEPTH:
        return _batched_grid_call(a_norm, f_norm, w0, w1,
                                  8 if b % 8 == 0 else 1)

    any_spec = pl.BlockSpec(memory_space=pltpu.MemorySpace.HBM)
    vmem = pl.BlockSpec(memory_space=pltpu.MemorySpace.VMEM)
    return pl.pallas_call(
        _pipelined_body,
        out_shape=jax.ShapeDtypeStruct((b, n, d), jnp.float32),
        in_specs=[any_spec, any_spec, vmem, vmem],
        out_specs=any_spec,
        scratch_shapes=[
            pltpu.VMEM((_DEPTH, _CHUNK, n, n), jnp.float32),
            pltpu.VMEM((_DEPTH, _CHUNK, n, d), jnp.float32),
            pltpu.VMEM((_DEPTH, _CHUNK, n, d), jnp.float32),
            pltpu.SemaphoreType.DMA((_DEPTH,)),
            pltpu.SemaphoreType.DMA((_DEPTH,)),
            pltpu.SemaphoreType.DMA((_DEPTH,)),
        ],
    )(a_norm, f_norm, w0, w1)
